# baseline (speedup 1.0000x reference)
import jax
import jax.numpy as jnp
from jax.experimental import pallas as pl

BN = 32_000
CB = BN // 128


def _net_block(zt_ref, w_ref, c_ref, o_ref):
    zb = zt_ref[...]
    d = jnp.dot(w_ref[...], zb, preferred_element_type=jnp.float32)
    dd = d.reshape(CB, 128) + c_ref[0, 0]
    sp = jax.nn.sigmoid(dd)
    o_ref[:, 0, :] = sp
    o_ref[:, 1, :] = 1.0 - sp


def kernel(z, W, b):
    wd = (W[0] - W[1]).reshape(1, 32)
    bd = (b[0] - b[1]).reshape(1, 1)
    n = z.shape[0]
    zt = z.T
    grid = (n // BN,)
    o = pl.pallas_call(
        _net_block,
        grid=grid,
        in_specs=[
            pl.BlockSpec((32, BN), lambda i: (0, i)),
            pl.BlockSpec((1, 32), lambda i: (0, 0)),
            pl.BlockSpec((1, 1), lambda i: (0, 0)),
        ],
        out_specs=pl.BlockSpec((CB, 2, 128), lambda i: (i, 0, 0)),
        out_shape=jax.ShapeDtypeStruct((n // 128, 2, 128), jnp.float32),
    )(zt, wd, bd)
    return o.transpose(0, 2, 1).reshape(n, 2)


# speedup vs baseline: 1.1182x; 1.1182x over previous
import jax
import jax.numpy as jnp
from jax.experimental import pallas as pl

BN = 80_000
CB = BN // 128


def _net_block(zt_ref, w_ref, c_ref, o_ref):
    zb = zt_ref[...]
    d = jnp.dot(w_ref[...], zb, preferred_element_type=jnp.float32)
    dd = d.reshape(CB, 128) + c_ref[0, 0]
    sp = jax.nn.sigmoid(dd)
    o_ref[:, 0, :] = sp
    o_ref[:, 1, :] = 1.0 - sp


def kernel(z, W, b):
    wd = (W[0] - W[1]).reshape(1, 32)
    bd = (b[0] - b[1]).reshape(1, 1)
    n = z.shape[0]
    zt = z.T
    grid = (n // BN,)
    o = pl.pallas_call(
        _net_block,
        grid=grid,
        in_specs=[
            pl.BlockSpec((32, BN), lambda i: (0, i)),
            pl.BlockSpec((1, 32), lambda i: (0, 0)),
            pl.BlockSpec((1, 1), lambda i: (0, 0)),
        ],
        out_specs=pl.BlockSpec((CB, 2, 128), lambda i: (i, 0, 0)),
        out_shape=jax.ShapeDtypeStruct((n // 128, 2, 128), jnp.float32),
    )(zt, wd, bd)
    return o.transpose(0, 2, 1).reshape(n, 2)


# speedup vs baseline: 1.1231x; 1.0045x over previous
import jax
import jax.numpy as jnp
from jax.experimental import pallas as pl
from jax.experimental.pallas import tpu as pltpu

BN = 64_000
CB = BN // 128


def _net_block(zt_ref, w_ref, c_ref, o_ref):
    zb = zt_ref[...]
    d = jnp.dot(w_ref[...], zb, preferred_element_type=jnp.float32)
    dd = d.reshape(CB, 128) + c_ref[0, 0]
    sp = jax.nn.sigmoid(dd)
    o_ref[:, 0, :] = sp
    o_ref[:, 1, :] = 1.0 - sp


def kernel(z, W, b):
    wd = (W[0] - W[1]).reshape(1, 32)
    bd = (b[0] - b[1]).reshape(1, 1)
    n = z.shape[0]
    zt = z.T
    grid = (n // BN,)
    o = pl.pallas_call(
        _net_block,
        grid=grid,
        in_specs=[
            pl.BlockSpec((32, BN), lambda i: (0, i)),
            pl.BlockSpec((1, 32), lambda i: (0, 0)),
            pl.BlockSpec((1, 1), lambda i: (0, 0)),
        ],
        out_specs=pl.BlockSpec((CB, 2, 128), lambda i: (i, 0, 0)),
        out_shape=jax.ShapeDtypeStruct((n // 128, 2, 128), jnp.float32),
        compiler_params=pltpu.CompilerParams(
            dimension_semantics=("parallel",),
        ),
    )(zt, wd, bd)
    return o.transpose(0, 2, 1).reshape(n, 2)
